# baseline (speedup 1.0000x reference)
import functools

import jax
import jax.numpy as jnp
from jax import lax
from jax.experimental import pallas as pl
from jax.experimental.pallas import tpu as pltpu
from jax.experimental.pallas import tpu_sc as plsc

N_NODES = 10000
N_EDGES = 320000
D_FEAT = 128
HIDDEN_DIM = 128
MAX_COLORS = 16
BATCH_NODES = 1024

NC = 2
NS = 16
L = 16

CHUNK = 128
E_T = N_EDGES // NS
N_SEG = 5
E_SEG = E_T // N_SEG
N_GRP = E_SEG // L
C_CAP = -(-E_SEG // CHUNK)
RH = 5120
DUMP = RH
RS = RH + CHUNK
RPT = RS // NS
R_PAD = NC * RH


_mesh = plsc.VectorSubcoreMesh(
    core_axis_name="c", subcore_axis_name="s", num_cores=NC, num_subcores=NS)


@functools.partial(
    pl.kernel,
    out_type=[
        jax.ShapeDtypeStruct((NC, RS, D_FEAT), jnp.float32),
        jax.ShapeDtypeStruct((NC, NS, RS), jnp.float32),
    ],
    mesh=_mesh,
    compiler_params=pltpu.CompilerParams(needs_layout_passes=False),
    scratch_types=[
        pltpu.VMEM((N_NODES,), jnp.int32),
        pltpu.VMEM((BATCH_NODES,), jnp.int32),
        pltpu.VMEM((E_SEG,), jnp.int32),
        pltpu.VMEM((E_SEG,), jnp.int32),
        pltpu.VMEM((E_SEG,), jnp.int32),
        pltpu.VMEM((E_SEG,), jnp.int32),
        pltpu.VMEM((C_CAP, CHUNK), jnp.int32),
        pltpu.VMEM((C_CAP, CHUNK), jnp.int32),
        pltpu.VMEM((RS,), jnp.float32),
        pltpu.VMEM((2, CHUNK, D_FEAT), jnp.float32),
        pltpu.VMEM_SHARED((RS, D_FEAT), jnp.float32),
        pltpu.SemaphoreType.DMA,
        pltpu.SemaphoreType.DMA,
        pltpu.SemaphoreType.DMA,
        pltpu.SemaphoreType.DMA,
        pltpu.SemaphoreType.DMA,
    ],
)
def _sc_aggregate(src_hbm, dst_hbm, cn_hbm, x_hbm, acc_out, deg_out,
                  sel_v, cn_v, src_e0, src_e1, dst_e0, dst_e1,
                  comp_src, comp_dst,
                  degl, rowbuf, acc_sh,
                  sem_s0, sem_s1, sem_g0, sem_g1, sem_z):
    cid = lax.axis_index("c")
    sid = lax.axis_index("s")

    zero16 = jnp.zeros((L,), jnp.float32)
    ones16 = jnp.ones((L,), jnp.float32)
    iota16 = lax.iota(jnp.int32, L)
    sem_s = (sem_s0, sem_s1)
    sem_g = (sem_g0, sem_g1)
    src_e = (src_e0, src_e1)
    dst_e = (dst_e0, dst_e1)

    def _fill(i, c):
        for j in range(D_FEAT // L):
            rowbuf[0, i, pl.ds(j * L, L)] = zero16
        return c

    lax.fori_loop(0, CHUNK, _fill, 0)

    row0 = sid * RPT
    zsegs = ((0, CHUNK), (CHUNK, CHUNK), (2 * CHUNK, RPT - 2 * CHUNK))
    for k0, sz in zsegs:
        pltpu.async_copy(rowbuf.at[0, pl.ds(0, sz)],
                         acc_sh.at[pl.ds(row0 + k0, sz)], sem_z)

    e0 = sid * E_T
    pltpu.async_copy(src_hbm.at[pl.ds(e0, E_SEG)], src_e[0], sem_s[0])
    pltpu.async_copy(dst_hbm.at[pl.ds(e0, E_SEG)], dst_e[0], sem_s[0])

    def _dz(i, c):
        degl[pl.ds(i * L, L)] = zero16
        return c

    lax.fori_loop(0, RS // L, _dz, 0)

    def _selz(i, c):
        sel_v[pl.ds(i * L, L)] = jnp.zeros((L,), jnp.int32)
        return c

    lax.fori_loop(0, N_NODES // L, _selz, 0)
    pltpu.sync_copy(cn_hbm, cn_v)

    def _sels(i, c):
        plsc.store_scatter(sel_v, [cn_v[pl.ds(i * L, L)]],
                           jnp.ones((L,), jnp.int32),
                           mask=jnp.full((L,), True))
        return c

    lax.fori_loop(0, BATCH_NODES // L, _sels, 0)

    dump16 = jnp.full((L,), DUMP, jnp.int32)

    for seg in range(N_SEG):
        pb = seg % 2
        pltpu.make_async_copy(src_hbm.at[pl.ds(e0 + seg * E_SEG, E_SEG)],
                              src_e[pb], sem_s[pb]).wait()
        pltpu.make_async_copy(dst_hbm.at[pl.ds(e0 + seg * E_SEG, E_SEG)],
                              dst_e[pb], sem_s[pb]).wait()
        if seg + 1 < N_SEG:
            off = e0 + (seg + 1) * E_SEG
            nb = (seg + 1) % 2
            pltpu.async_copy(src_hbm.at[pl.ds(off, E_SEG)],
                             src_e[nb], sem_s[nb])
            pltpu.async_copy(dst_hbm.at[pl.ds(off, E_SEG)],
                             dst_e[nb], sem_s[nb])

        def _grp(i, cur):
            s16 = src_e[pb][pl.ds(i * L, L)]
            d16 = dst_e[pb][pl.ds(i * L, L)]
            selv = plsc.load_gather(sel_v, [d16])
            dloc = d16 - cid * RH
            m = (selv > 0) & (dloc >= 0) & (dloc < RH)
            inc = plsc.cumsum(m.astype(jnp.int32))
            pos = cur + inc - 1
            plsc.store_scatter(comp_src, [pos >> 7, pos & (CHUNK - 1)],
                               s16, mask=m)
            plsc.store_scatter(comp_dst, [pos >> 7, pos & (CHUNK - 1)],
                               dloc, mask=m)
            plsc.addupdate_scatter(degl, [dloc], ones16, mask=m)
            return cur + plsc.all_reduce_population_count(m)

        cnt_v = lax.fori_loop(0, N_GRP, _grp, jnp.zeros((L,), jnp.int32))

        cend_v = (cnt_v + CHUNK - 1) & ~(CHUNK - 1)
        for g in range(CHUNK // L):
            p = cnt_v + g * L + iota16
            mm = p < cend_v
            plsc.store_scatter(comp_src, [p >> 7, p & (CHUNK - 1)],
                               jnp.zeros((L,), jnp.int32), mask=mm)
            plsc.store_scatter(comp_dst, [p >> 7, p & (CHUNK - 1)],
                               dump16, mask=mm)
        nch = jnp.max(cend_v) >> 7

        if seg == 0:
            for k0, sz in zsegs:
                pltpu.make_async_copy(
                    rowbuf.at[0, pl.ds(0, sz)],
                    acc_sh.at[pl.ds(row0 + k0, sz)], sem_z).wait()

            plsc.subcore_barrier()

        @pl.when(nch > 0)
        def _prologue():
            pltpu.async_copy(x_hbm.at[comp_src.at[0]], rowbuf.at[0],
                             sem_g[0])

        def _chunk(j, c):
            b = j & 1

            @pl.when(j + 1 < nch)
            def _fire_next():
                for nb in range(2):
                    @pl.when((1 - b) == nb)
                    def _():
                        pltpu.async_copy(x_hbm.at[comp_src.at[j + 1]],
                                         rowbuf.at[nb], sem_g[nb])

            for bb in range(2):
                @pl.when(b == bb)
                def _():
                    pltpu.make_async_copy(x_hbm.at[comp_src.at[j]],
                                          rowbuf.at[bb], sem_g[bb]).wait()
                    pltpu.sync_copy(rowbuf.at[bb],
                                    acc_sh.at[comp_dst.at[j]], add=True)
            return c

        lax.fori_loop(0, nch, _chunk, 0)

    pltpu.sync_copy(degl, deg_out.at[cid, sid])
    plsc.subcore_barrier()
    pltpu.sync_copy(acc_sh.at[pl.ds(row0, RPT)],
                    acc_out.at[cid, pl.ds(row0, RPT)])


_RB = 1024
_NB = RH // _RB


def _tc_dense_body(acc_ref, deg_ref, wg_ref, bg_ref, wf_ref, bf_ref, out_ref):
    a = acc_ref[0]
    d = deg_ref[0, 0]
    for k in range(1, NS):
        d = d + deg_ref[0, k]
    e = jnp.maximum(
        jnp.dot(a / jnp.maximum(d, 1.0), wg_ref[...],
                preferred_element_type=jnp.float32)
        + bg_ref[...][None, :], 0.0)
    f = (jnp.dot(e, wf_ref[...], preferred_element_type=jnp.float32)
         + bf_ref[...][None, :])
    out_ref[...] = jnp.concatenate(
        [f, jnp.zeros((_RB, D_FEAT - MAX_COLORS), jnp.float32)], axis=1)


def _tc_dense(acc, deg, W_gnn, b_gnn, W_fc, b_fc):
    return pl.pallas_call(
        _tc_dense_body,
        grid=(NC * _NB,),
        in_specs=[
            pl.BlockSpec((1, _RB, D_FEAT), lambda i: (i // _NB, i % _NB, 0)),
            pl.BlockSpec((1, NS, _RB, 1),
                         lambda i: (i // _NB, 0, i % _NB, 0)),
            pl.BlockSpec((D_FEAT, HIDDEN_DIM), lambda i: (0, 0)),
            pl.BlockSpec((HIDDEN_DIM,), lambda i: (0,)),
            pl.BlockSpec((HIDDEN_DIM, MAX_COLORS), lambda i: (0, 0)),
            pl.BlockSpec((MAX_COLORS,), lambda i: (0,)),
        ],
        out_specs=pl.BlockSpec((_RB, D_FEAT), lambda i: (i, 0)),
        out_shape=jax.ShapeDtypeStruct((R_PAD, D_FEAT), jnp.float32),
    )(acc, deg.reshape(NC, NS, RS, 1), W_gnn, b_gnn, W_fc, b_fc)


_B_W = BATCH_NODES // (NC * NS)


@functools.partial(
    pl.kernel,
    out_type=jax.ShapeDtypeStruct((BATCH_NODES, D_FEAT), jnp.float32),
    mesh=_mesh,
    compiler_params=pltpu.CompilerParams(needs_layout_passes=False),
    scratch_types=[
        pltpu.VMEM((_B_W,), jnp.int32),
        pltpu.VMEM((_B_W, D_FEAT), jnp.float32),
        pltpu.SemaphoreType.DMA,
    ],
)
def _sc_select(f_hbm, cn_hbm, out_hbm, idx_v, rows_v, sem):
    wid = lax.axis_index("c") * NS + lax.axis_index("s")
    base = wid * _B_W
    pltpu.sync_copy(cn_hbm.at[pl.ds(base, _B_W)], idx_v)
    pltpu.async_copy(f_hbm.at[idx_v], rows_v, sem).wait()
    pltpu.sync_copy(rows_v, out_hbm.at[pl.ds(base, _B_W)])


def kernel(x, edge_index, current_node, W_gnn, b_gnn, W_fc, b_fc):
    acc, deg = _sc_aggregate(edge_index[0], edge_index[1], current_node, x)
    logits = _tc_dense(acc, deg, W_gnn, b_gnn, W_fc, b_fc)
    return _sc_select(logits, current_node)[:, :MAX_COLORS]


# speedup vs baseline: 1.1461x; 1.1461x over previous
import functools

import jax
import jax.numpy as jnp
from jax import lax
from jax.experimental import pallas as pl
from jax.experimental.pallas import tpu as pltpu
from jax.experimental.pallas import tpu_sc as plsc

N_NODES = 10000
N_EDGES = 320000
D_FEAT = 128
HIDDEN_DIM = 128
MAX_COLORS = 16
BATCH_NODES = 1024

NC = 2
NS = 16
L = 16

CHUNK = 128
E_T = N_EDGES // NS
N_SEG = 5
E_SEG = E_T // N_SEG
N_GRP = E_SEG // L
C_CAP = -(-E_SEG // CHUNK)
RH = 5120
DUMP = RH
RS = RH + CHUNK
RPT = RS // NS
R_PAD = NC * RH


_mesh = plsc.VectorSubcoreMesh(
    core_axis_name="c", subcore_axis_name="s", num_cores=NC, num_subcores=NS)


@functools.partial(
    pl.kernel,
    out_type=[
        jax.ShapeDtypeStruct((NC, RS, D_FEAT), jnp.float32),
        jax.ShapeDtypeStruct((NC, NS, RS), jnp.float32),
    ],
    mesh=_mesh,
    compiler_params=pltpu.CompilerParams(needs_layout_passes=False),
    scratch_types=[
        pltpu.VMEM((N_NODES,), jnp.int32),
        pltpu.VMEM((BATCH_NODES,), jnp.int32),
        pltpu.VMEM((E_SEG,), jnp.int32),
        pltpu.VMEM((E_SEG,), jnp.int32),
        pltpu.VMEM((E_SEG,), jnp.int32),
        pltpu.VMEM((E_SEG,), jnp.int32),
        pltpu.VMEM((C_CAP, CHUNK), jnp.int32),
        pltpu.VMEM((C_CAP, CHUNK), jnp.int32),
        pltpu.VMEM((RS,), jnp.float32),
        pltpu.VMEM((2, CHUNK, D_FEAT), jnp.float32),
        pltpu.VMEM_SHARED((RS, D_FEAT), jnp.float32),
        pltpu.SemaphoreType.DMA,
        pltpu.SemaphoreType.DMA,
        pltpu.SemaphoreType.DMA,
        pltpu.SemaphoreType.DMA,
        pltpu.SemaphoreType.DMA,
    ],
)
def _sc_aggregate(src_hbm, dst_hbm, cn_hbm, x_hbm, acc_out, deg_out,
                  sel_v, cn_v, src_e0, src_e1, dst_e0, dst_e1,
                  comp_src, comp_dst,
                  degl, rowbuf, acc_sh,
                  sem_s0, sem_s1, sem_g0, sem_g1, sem_z):
    cid = lax.axis_index("c")
    sid = lax.axis_index("s")

    zero16 = jnp.zeros((L,), jnp.float32)
    ones16 = jnp.ones((L,), jnp.float32)
    iota16 = lax.iota(jnp.int32, L)
    sem_s = (sem_s0, sem_s1)
    sem_g = (sem_g0, sem_g1)
    src_e = (src_e0, src_e1)
    dst_e = (dst_e0, dst_e1)

    @plsc.parallel_loop(0, CHUNK, unroll=4)
    def _fill(i):
        for j in range(D_FEAT // L):
            rowbuf[0, i, pl.ds(j * L, L)] = zero16

    row0 = sid * RPT
    zsegs = ((0, CHUNK), (CHUNK, CHUNK), (2 * CHUNK, RPT - 2 * CHUNK))
    for k0, sz in zsegs:
        pltpu.async_copy(rowbuf.at[0, pl.ds(0, sz)],
                         acc_sh.at[pl.ds(row0 + k0, sz)], sem_z)

    e0 = sid * E_T
    pltpu.async_copy(src_hbm.at[pl.ds(e0, E_SEG)], src_e[0], sem_s[0])
    pltpu.async_copy(dst_hbm.at[pl.ds(e0, E_SEG)], dst_e[0], sem_s[0])

    @plsc.parallel_loop(0, RS // L, unroll=4)
    def _dz(i):
        degl[pl.ds(i * L, L)] = zero16

    @plsc.parallel_loop(0, N_NODES // L, unroll=4)
    def _selz(i):
        sel_v[pl.ds(i * L, L)] = jnp.zeros((L,), jnp.int32)

    pltpu.sync_copy(cn_hbm, cn_v)

    @plsc.parallel_loop(0, BATCH_NODES // L, unroll=4)
    def _sels(i):
        plsc.store_scatter(sel_v, [cn_v[pl.ds(i * L, L)]],
                           jnp.ones((L,), jnp.int32),
                           mask=jnp.full((L,), True))

    dump16 = jnp.full((L,), DUMP, jnp.int32)

    for seg in range(N_SEG):
        pb = seg % 2
        pltpu.make_async_copy(src_hbm.at[pl.ds(e0 + seg * E_SEG, E_SEG)],
                              src_e[pb], sem_s[pb]).wait()
        pltpu.make_async_copy(dst_hbm.at[pl.ds(e0 + seg * E_SEG, E_SEG)],
                              dst_e[pb], sem_s[pb]).wait()
        if seg + 1 < N_SEG:
            off = e0 + (seg + 1) * E_SEG
            nb = (seg + 1) % 2
            pltpu.async_copy(src_hbm.at[pl.ds(off, E_SEG)],
                             src_e[nb], sem_s[nb])
            pltpu.async_copy(dst_hbm.at[pl.ds(off, E_SEG)],
                             dst_e[nb], sem_s[nb])

        @plsc.parallel_loop(0, N_GRP, unroll=4,
                            carry=jnp.zeros((L,), jnp.int32))
        def _grp(i, cur):
            s16 = src_e[pb][pl.ds(i * L, L)]
            d16 = dst_e[pb][pl.ds(i * L, L)]
            selv = plsc.load_gather(sel_v, [d16])
            dloc = d16 - cid * RH
            m = (selv > 0) & (dloc >= 0) & (dloc < RH)
            inc = plsc.cumsum(m.astype(jnp.int32))
            pos = cur + inc - 1
            plsc.store_scatter(comp_src, [pos >> 7, pos & (CHUNK - 1)],
                               s16, mask=m)
            plsc.store_scatter(comp_dst, [pos >> 7, pos & (CHUNK - 1)],
                               dloc, mask=m)
            plsc.addupdate_scatter(degl, [dloc], ones16, mask=m)
            return cur + plsc.all_reduce_population_count(m)

        cnt_v = _grp

        cend_v = (cnt_v + CHUNK - 1) & ~(CHUNK - 1)
        for g in range(CHUNK // L):
            p = cnt_v + g * L + iota16
            mm = p < cend_v
            plsc.store_scatter(comp_src, [p >> 7, p & (CHUNK - 1)],
                               jnp.zeros((L,), jnp.int32), mask=mm)
            plsc.store_scatter(comp_dst, [p >> 7, p & (CHUNK - 1)],
                               dump16, mask=mm)
        nch = jnp.max(cend_v) >> 7

        if seg == 0:
            for k0, sz in zsegs:
                pltpu.make_async_copy(
                    rowbuf.at[0, pl.ds(0, sz)],
                    acc_sh.at[pl.ds(row0 + k0, sz)], sem_z).wait()

            plsc.subcore_barrier()

        @pl.when(nch > 0)
        def _prologue():
            pltpu.async_copy(x_hbm.at[comp_src.at[0]], rowbuf.at[0],
                             sem_g[0])

        def _chunk(j, c):
            b = j & 1

            @pl.when(j + 1 < nch)
            def _fire_next():
                for nb in range(2):
                    @pl.when((1 - b) == nb)
                    def _():
                        pltpu.async_copy(x_hbm.at[comp_src.at[j + 1]],
                                         rowbuf.at[nb], sem_g[nb])

            for bb in range(2):
                @pl.when(b == bb)
                def _():
                    pltpu.make_async_copy(x_hbm.at[comp_src.at[j]],
                                          rowbuf.at[bb], sem_g[bb]).wait()
                    pltpu.sync_copy(rowbuf.at[bb],
                                    acc_sh.at[comp_dst.at[j]], add=True)
            return c

        lax.fori_loop(0, nch, _chunk, 0)

    pltpu.sync_copy(degl, deg_out.at[cid, sid])
    plsc.subcore_barrier()
    pltpu.sync_copy(acc_sh.at[pl.ds(row0, RPT)],
                    acc_out.at[cid, pl.ds(row0, RPT)])


_RB = 1024
_NB = RH // _RB


def _tc_dense_body(acc_ref, deg_ref, wg_ref, bg_ref, wf_ref, bf_ref, out_ref):
    a = acc_ref[0]
    d = jnp.sum(deg_ref[0], axis=1, keepdims=True)
    e = jnp.maximum(
        jnp.dot(a / jnp.maximum(d, 1.0), wg_ref[...],
                preferred_element_type=jnp.float32)
        + bg_ref[...][None, :], 0.0)
    f = (jnp.dot(e, wf_ref[...], preferred_element_type=jnp.float32)
         + bf_ref[...][None, :])
    out_ref[...] = jnp.concatenate(
        [f, jnp.zeros((_RB, D_FEAT - MAX_COLORS), jnp.float32)], axis=1)


def _tc_dense(acc, deg, W_gnn, b_gnn, W_fc, b_fc):
    return pl.pallas_call(
        _tc_dense_body,
        grid=(NC * _NB,),
        in_specs=[
            pl.BlockSpec((1, _RB, D_FEAT), lambda i: (i // _NB, i % _NB, 0)),
            pl.BlockSpec((1, _RB, NS), lambda i: (i // _NB, i % _NB, 0)),
            pl.BlockSpec((D_FEAT, HIDDEN_DIM), lambda i: (0, 0)),
            pl.BlockSpec((HIDDEN_DIM,), lambda i: (0,)),
            pl.BlockSpec((HIDDEN_DIM, MAX_COLORS), lambda i: (0, 0)),
            pl.BlockSpec((MAX_COLORS,), lambda i: (0,)),
        ],
        out_specs=pl.BlockSpec((_RB, D_FEAT), lambda i: (i, 0)),
        out_shape=jax.ShapeDtypeStruct((R_PAD, D_FEAT), jnp.float32),
    )(acc, deg.transpose(0, 2, 1), W_gnn, b_gnn, W_fc, b_fc)


_B_W = BATCH_NODES // (NC * NS)


@functools.partial(
    pl.kernel,
    out_type=jax.ShapeDtypeStruct((BATCH_NODES, D_FEAT), jnp.float32),
    mesh=_mesh,
    compiler_params=pltpu.CompilerParams(needs_layout_passes=False),
    scratch_types=[
        pltpu.VMEM((_B_W,), jnp.int32),
        pltpu.VMEM((_B_W, D_FEAT), jnp.float32),
        pltpu.SemaphoreType.DMA,
    ],
)
def _sc_select(f_hbm, cn_hbm, out_hbm, idx_v, rows_v, sem):
    wid = lax.axis_index("c") * NS + lax.axis_index("s")
    base = wid * _B_W
    pltpu.sync_copy(cn_hbm.at[pl.ds(base, _B_W)], idx_v)
    pltpu.async_copy(f_hbm.at[idx_v], rows_v, sem).wait()
    pltpu.sync_copy(rows_v, out_hbm.at[pl.ds(base, _B_W)])


def kernel(x, edge_index, current_node, W_gnn, b_gnn, W_fc, b_fc):
    acc, deg = _sc_aggregate(edge_index[0], edge_index[1], current_node, x)
    logits = _tc_dense(acc, deg, W_gnn, b_gnn, W_fc, b_fc)
    return _sc_select(logits, current_node)[:, :MAX_COLORS]
